# Initial kernel scaffold; baseline (speedup 1.0000x reference)
import jax
import jax.numpy as jnp
from jax.experimental import pallas as pl


def kernel(xyz, xyz_goal, params):
    raise NotImplementedError("write your pallas kernel here")


# speedup vs baseline: 1.1256x; 1.1256x over previous
import functools

import jax
import jax.numpy as jnp
from jax.experimental import pallas as pl


def _head_kernel(x_ref, w1_ref, b1_ref, g_ref, be_ref, w2_ref, b2_ref, o_ref):
    x = x_ref[...]
    y = jnp.dot(x, w1_ref[...].T, preferred_element_type=jnp.float32)
    y = y + b1_ref[...][None, :]
    mean = jnp.mean(y, axis=1, keepdims=True)
    var = jnp.mean((y - mean) ** 2, axis=1, keepdims=True)
    y = (y - mean) / jnp.sqrt(var + 1e-5)
    y = y * g_ref[...][None, :] + be_ref[...][None, :]
    y = jax.nn.relu(y)
    z = jnp.dot(y, w2_ref[...].T, preferred_element_type=jnp.float32)
    z = z + b2_ref[...][None, :]
    z = z - jax.scipy.special.logsumexp(z, axis=1, keepdims=True)
    o_ref[...] = jnp.broadcast_to(z[:, :, None], o_ref.shape)


def _head(xcat, params):
    B = xcat.shape[0]
    return pl.pallas_call(
        _head_kernel,
        out_shape=jax.ShapeDtypeStruct((B, 2, 1024), jnp.float32),
    )(xcat, params['conv1_w'], params['conv1_b'], params['gn_g'],
      params['gn_b'], params['conv2_w'], params['conv2_b'])


def _square_distance(src, dst):
    dist = -2.0 * jnp.matmul(src, jnp.swapaxes(dst, 1, 2))
    dist = dist + jnp.sum(src ** 2, -1)[:, :, None]
    dist = dist + jnp.sum(dst ** 2, -1)[:, None, :]
    return dist


def _index_points(points, idx):
    return jax.vmap(lambda p, i: p[i])(points, idx)


def _farthest_point_sample(xyz, npoint):
    B, N, C = xyz.shape
    def body(i, state):
        centroids, distance, farthest = state
        centroids = centroids.at[:, i].set(farthest)
        centroid = jnp.take_along_axis(xyz, farthest[:, None, None], axis=1)
        d = jnp.sum((xyz - centroid) ** 2, -1)
        distance = jnp.minimum(distance, d)
        farthest = jnp.argmax(distance, axis=-1).astype(jnp.int32)
        return (centroids, distance, farthest)
    init = (jnp.zeros((B, npoint), jnp.int32),
            jnp.full((B, N), 1e10, xyz.dtype),
            jnp.zeros((B,), jnp.int32))
    centroids, _, _ = jax.lax.fori_loop(0, npoint, body, init)
    return centroids


def _compute_density(xyz, bandwidth):
    sqrdists = _square_distance(xyz, xyz)
    g = jnp.exp(-sqrdists / (2.0 * bandwidth * bandwidth)) / (2.5 * bandwidth)
    return jnp.mean(g, axis=-1)


def _bn_eval(x):
    return x / jnp.sqrt(1.0 + 1e-5)


def _conv1x1(x, w, b):
    return jnp.einsum('oi,bihw->bohw', w, x) + b[None, :, None, None]


def _weightnet(x, wn):
    for (cw, cb) in wn:
        x = jax.nn.relu(_bn_eval(_conv1x1(x, cw, cb)))
    return x


def _pointconv_sa(xyz, points, p, npoint, nsample, bandwidth, group_all):
    B, _, N = xyz.shape
    xyz_t = jnp.swapaxes(xyz, 1, 2)
    pts_t = jnp.swapaxes(points, 1, 2)
    inv_density = (1.0 / _compute_density(xyz_t, bandwidth))[:, :, None]
    if group_all:
        new_xyz = jnp.zeros((B, 1, 3), xyz.dtype)
        grouped_xyz_norm = xyz_t[:, None, :, :]
        new_points = jnp.concatenate([grouped_xyz_norm, pts_t[:, None, :, :]], -1)
        grouped_density = inv_density[:, None, :, :]
        S = 1
    else:
        fps_idx = _farthest_point_sample(xyz_t, npoint)
        new_xyz = _index_points(xyz_t, fps_idx)
        sqrd = _square_distance(new_xyz, xyz_t)
        _, idx = jax.lax.top_k(-sqrd, nsample)
        grouped_xyz = _index_points(xyz_t, idx)
        grouped_xyz_norm = grouped_xyz - new_xyz[:, :, None, :]
        new_points = jnp.concatenate([grouped_xyz_norm, _index_points(pts_t, idx)], -1)
        grouped_density = _index_points(inv_density, idx)
        S = npoint
    new_points = jnp.transpose(new_points, (0, 3, 2, 1))
    new_points = jax.nn.relu(_bn_eval(_conv1x1(new_points, p['conv_w'], p['conv_b'])))
    density_scale = grouped_density / jnp.max(grouped_density, axis=2, keepdims=True)
    new_points = new_points * jnp.transpose(density_scale, (0, 3, 2, 1))
    weights = _weightnet(jnp.transpose(grouped_xyz_norm, (0, 3, 2, 1)), p['wn'])
    out = jnp.matmul(jnp.transpose(new_points, (0, 3, 1, 2)),
                     jnp.transpose(weights, (0, 3, 2, 1))).reshape(B, S, -1)
    out = out @ p['lin_w'].T + p['lin_b']
    out = jax.nn.relu(_bn_eval(jnp.swapaxes(out, 1, 2)))
    return jnp.swapaxes(new_xyz, 1, 2), out


def _encode(xyz, params):
    l1_xyz, l1_points = _pointconv_sa(xyz, xyz, params['sa1'], 512, 32, 0.1, False)
    l2_xyz, l2_points = _pointconv_sa(l1_xyz, l1_points, params['sa2'], 128, 64, 0.2, False)
    _, l3_points = _pointconv_sa(l2_xyz, l2_points, params['sa3'], 1, None, 0.4, True)
    return l3_points.reshape(xyz.shape[0], 256)


@jax.jit
def _run(xyz, xyz_goal, params):
    B = xyz.shape[0]
    both = jnp.concatenate([xyz, xyz_goal], axis=0)
    feats = _encode(both, params)
    xcat = jnp.concatenate([feats[:B], feats[B:]], axis=-1)
    return _head(xcat, params)


def kernel(xyz, xyz_goal, params):
    return _run(xyz, xyz_goal, params)


# speedup vs baseline: 1.3671x; 1.2146x over previous
import functools

import jax
import jax.numpy as jnp
from jax.experimental import pallas as pl


def _head_kernel(x_ref, w1_ref, b1_ref, g_ref, be_ref, w2_ref, b2_ref, o_ref):
    x = x_ref[...]
    y = jnp.dot(x, w1_ref[...].T, preferred_element_type=jnp.float32)
    y = y + b1_ref[...][None, :]
    mean = jnp.mean(y, axis=1, keepdims=True)
    var = jnp.mean((y - mean) ** 2, axis=1, keepdims=True)
    y = (y - mean) / jnp.sqrt(var + 1e-5)
    y = y * g_ref[...][None, :] + be_ref[...][None, :]
    y = jax.nn.relu(y)
    z = jnp.dot(y, w2_ref[...].T, preferred_element_type=jnp.float32)
    z = z + b2_ref[...][None, :]
    z = z - jax.scipy.special.logsumexp(z, axis=1, keepdims=True)
    o_ref[...] = jnp.broadcast_to(z[:, :, None], o_ref.shape)


def _head(xcat, params):
    B = xcat.shape[0]
    return pl.pallas_call(
        _head_kernel,
        out_shape=jax.ShapeDtypeStruct((B, 2, 1024), jnp.float32),
    )(xcat, params['conv1_w'], params['conv1_b'], params['gn_g'],
      params['gn_b'], params['conv2_w'], params['conv2_b'])


def _fps_kernel(npoint, x_ref, y_ref, z_ref, o_ref):
    x = x_ref[...]
    y = y_ref[...]
    z = z_ref[...]
    B, N = x.shape
    iota = jax.lax.broadcasted_iota(jnp.int32, (B, N), 1)
    cols = jax.lax.broadcasted_iota(jnp.int32, o_ref.shape, 1)

    def body(i, carry):
        distance, fidx, cx, cy, cz = carry
        o_ref[...] = jnp.where(cols == i, fidx, o_ref[...])
        d = (x - cx) ** 2 + (y - cy) ** 2 + (z - cz) ** 2
        distance = jnp.minimum(distance, d)
        m = jnp.max(distance, axis=1, keepdims=True)
        idx = jnp.min(jnp.where(distance == m, iota, N), axis=1, keepdims=True)
        onehot = iota == idx
        zero = jnp.zeros((), jnp.float32)
        cx = jnp.sum(jnp.where(onehot, x, zero), axis=1, keepdims=True)
        cy = jnp.sum(jnp.where(onehot, y, zero), axis=1, keepdims=True)
        cz = jnp.sum(jnp.where(onehot, z, zero), axis=1, keepdims=True)
        return (distance, idx, cx, cy, cz)

    o_ref[...] = jnp.zeros(o_ref.shape, jnp.int32)
    init = (jnp.full((B, N), 1e10, jnp.float32),
            jnp.zeros((B, 1), jnp.int32),
            x[:, 0:1], y[:, 0:1], z[:, 0:1])
    jax.lax.fori_loop(0, npoint, body, init)


def _fps(xyz_t, npoint):
    B, N, _ = xyz_t.shape
    x = xyz_t[:, :, 0]
    y = xyz_t[:, :, 1]
    z = xyz_t[:, :, 2]
    return pl.pallas_call(
        functools.partial(_fps_kernel, npoint),
        out_shape=jax.ShapeDtypeStruct((B, npoint), jnp.int32),
    )(x, y, z)


def _square_distance(src, dst):
    dist = -2.0 * jnp.matmul(src, jnp.swapaxes(dst, 1, 2))
    dist = dist + jnp.sum(src ** 2, -1)[:, :, None]
    dist = dist + jnp.sum(dst ** 2, -1)[:, None, :]
    return dist


def _index_points(points, idx):
    return jax.vmap(lambda p, i: p[i])(points, idx)


def _farthest_point_sample(xyz, npoint):
    B, N, C = xyz.shape
    def body(i, state):
        centroids, distance, farthest = state
        centroids = centroids.at[:, i].set(farthest)
        centroid = jnp.take_along_axis(xyz, farthest[:, None, None], axis=1)
        d = jnp.sum((xyz - centroid) ** 2, -1)
        distance = jnp.minimum(distance, d)
        farthest = jnp.argmax(distance, axis=-1).astype(jnp.int32)
        return (centroids, distance, farthest)
    init = (jnp.zeros((B, npoint), jnp.int32),
            jnp.full((B, N), 1e10, xyz.dtype),
            jnp.zeros((B,), jnp.int32))
    centroids, _, _ = jax.lax.fori_loop(0, npoint, body, init)
    return centroids


def _compute_density(xyz, bandwidth):
    sqrdists = _square_distance(xyz, xyz)
    g = jnp.exp(-sqrdists / (2.0 * bandwidth * bandwidth)) / (2.5 * bandwidth)
    return jnp.mean(g, axis=-1)


def _bn_eval(x):
    return x / jnp.sqrt(1.0 + 1e-5)


def _conv1x1(x, w, b):
    return jnp.einsum('oi,bihw->bohw', w, x) + b[None, :, None, None]


def _weightnet(x, wn):
    for (cw, cb) in wn:
        x = jax.nn.relu(_bn_eval(_conv1x1(x, cw, cb)))
    return x


def _pointconv_sa(xyz, points, p, npoint, nsample, bandwidth, group_all):
    B, _, N = xyz.shape
    xyz_t = jnp.swapaxes(xyz, 1, 2)
    pts_t = jnp.swapaxes(points, 1, 2)
    inv_density = (1.0 / _compute_density(xyz_t, bandwidth))[:, :, None]
    if group_all:
        new_xyz = jnp.zeros((B, 1, 3), xyz.dtype)
        grouped_xyz_norm = xyz_t[:, None, :, :]
        new_points = jnp.concatenate([grouped_xyz_norm, pts_t[:, None, :, :]], -1)
        grouped_density = inv_density[:, None, :, :]
        S = 1
    else:
        fps_idx = _fps(xyz_t, npoint)
        new_xyz = _index_points(xyz_t, fps_idx)
        sqrd = _square_distance(new_xyz, xyz_t)
        _, idx = jax.lax.top_k(-sqrd, nsample)
        grouped_xyz = _index_points(xyz_t, idx)
        grouped_xyz_norm = grouped_xyz - new_xyz[:, :, None, :]
        new_points = jnp.concatenate([grouped_xyz_norm, _index_points(pts_t, idx)], -1)
        grouped_density = _index_points(inv_density, idx)
        S = npoint
    new_points = jnp.transpose(new_points, (0, 3, 2, 1))
    new_points = jax.nn.relu(_bn_eval(_conv1x1(new_points, p['conv_w'], p['conv_b'])))
    density_scale = grouped_density / jnp.max(grouped_density, axis=2, keepdims=True)
    new_points = new_points * jnp.transpose(density_scale, (0, 3, 2, 1))
    weights = _weightnet(jnp.transpose(grouped_xyz_norm, (0, 3, 2, 1)), p['wn'])
    out = jnp.matmul(jnp.transpose(new_points, (0, 3, 1, 2)),
                     jnp.transpose(weights, (0, 3, 2, 1))).reshape(B, S, -1)
    out = out @ p['lin_w'].T + p['lin_b']
    out = jax.nn.relu(_bn_eval(jnp.swapaxes(out, 1, 2)))
    return jnp.swapaxes(new_xyz, 1, 2), out


def _encode(xyz, params):
    l1_xyz, l1_points = _pointconv_sa(xyz, xyz, params['sa1'], 512, 32, 0.1, False)
    l2_xyz, l2_points = _pointconv_sa(l1_xyz, l1_points, params['sa2'], 128, 64, 0.2, False)
    _, l3_points = _pointconv_sa(l2_xyz, l2_points, params['sa3'], 1, None, 0.4, True)
    return l3_points.reshape(xyz.shape[0], 256)


@jax.jit
def _run(xyz, xyz_goal, params):
    B = xyz.shape[0]
    both = jnp.concatenate([xyz, xyz_goal], axis=0)
    feats = _encode(both, params)
    xcat = jnp.concatenate([feats[:B], feats[B:]], axis=-1)
    return _head(xcat, params)


def kernel(xyz, xyz_goal, params):
    return _run(xyz, xyz_goal, params)


# speedup vs baseline: 6.3628x; 4.6541x over previous
import functools

import jax
import jax.numpy as jnp
from jax.experimental import pallas as pl

import numpy as _np
_BN = float(1.0 / _np.sqrt(_np.float32(1.0) + _np.float32(1e-5),
                           dtype=_np.float32))


def _fps_kernel(npoint, x_ref, y_ref, z_ref, o_ref):
    x = x_ref[...]
    y = y_ref[...]
    z = z_ref[...]
    B, N = x.shape
    iota = jax.lax.broadcasted_iota(jnp.int32, (B, N), 1)
    cols = jax.lax.broadcasted_iota(jnp.int32, o_ref.shape, 1)

    def body(i, carry):
        distance, fidx, cx, cy, cz = carry
        o_ref[...] = jnp.where(cols == i, fidx, o_ref[...])
        d = (x - cx) ** 2 + (y - cy) ** 2 + (z - cz) ** 2
        distance = jnp.minimum(distance, d)
        m = jnp.max(distance, axis=1, keepdims=True)
        idx = jnp.min(jnp.where(distance == m, iota, N), axis=1, keepdims=True)
        onehot = iota == idx
        zero = jnp.zeros((), jnp.float32)
        cx = jnp.sum(jnp.where(onehot, x, zero), axis=1, keepdims=True)
        cy = jnp.sum(jnp.where(onehot, y, zero), axis=1, keepdims=True)
        cz = jnp.sum(jnp.where(onehot, z, zero), axis=1, keepdims=True)
        return (distance, idx, cx, cy, cz)

    o_ref[...] = jnp.zeros(o_ref.shape, jnp.int32)
    init = (jnp.full((B, N), 1e10, jnp.float32),
            jnp.zeros((B, 1), jnp.int32),
            x[:, 0:1], y[:, 0:1], z[:, 0:1])
    jax.lax.fori_loop(0, npoint, body, init)


def _fps(xyz_t, npoint):
    B, N, _ = xyz_t.shape
    return pl.pallas_call(
        functools.partial(_fps_kernel, npoint),
        out_shape=jax.ShapeDtypeStruct((B, npoint), jnp.int32),
    )(xyz_t[:, :, 0], xyz_t[:, :, 1], xyz_t[:, :, 2])


def _square_distance(src, dst):
    dist = -2.0 * jnp.matmul(src, jnp.swapaxes(dst, 1, 2))
    dist = dist + jnp.sum(src ** 2, -1)[:, :, None]
    dist = dist + jnp.sum(dst ** 2, -1)[:, None, :]
    return dist


def _index_points(points, idx):
    return jax.vmap(lambda p, i: p[i])(points, idx)


def _inv_density(xyz_t, bandwidth):
    sqrdists = _square_distance(xyz_t, xyz_t)
    g = jnp.exp(-sqrdists / (2.0 * bandwidth * bandwidth)) / (2.5 * bandwidth)
    return 1.0 / jnp.mean(g, axis=-1)


def _sa_kernel(K, Cpts, p_ref, idx_ref, nx_ref,
               cwa_ref, cwb_ref, cb_ref,
               w1_ref, b1_ref, w2_ref, b2_ref, w3_ref, b3_ref,
               lw_ref, lb_ref, o_ref):
    N = p_ref.shape[1]
    Sb = idx_ref.shape[1]
    Cout = cb_ref.shape[1]

    P = p_ref[...][0]
    idx = idx_ref[...].reshape(Sb, K, 1)
    nx = nx_ref[...][0]

    iota = jax.lax.broadcasted_iota(jnp.int32, (Sb, K, N), 2)
    oh = (iota == idx).astype(jnp.float32).reshape(Sb * K, N)
    G = jnp.dot(oh, P, preferred_element_type=jnp.float32)

    gxyz = G[:, 0:3]
    gpts = G[:, 3:3 + Cpts]
    gden = G[:, 3 + Cpts:4 + Cpts]

    nxr = jnp.broadcast_to(nx[:, None, :], (Sb, K, 3)).reshape(Sb * K, 3)
    gn = gxyz - nxr

    A = jnp.dot(gn, cwa_ref[...], preferred_element_type=jnp.float32)
    A = A + jnp.dot(gpts, cwb_ref[...], preferred_element_type=jnp.float32)
    A = jax.nn.relu((A + cb_ref[...]) * _BN)

    gden3 = gden.reshape(Sb, K, 1)
    scale = gden3 / jnp.max(gden3, axis=1, keepdims=True)
    A3 = A.reshape(Sb, K, Cout) * scale

    h = jax.nn.relu((jnp.dot(gn, w1_ref[...],
                             preferred_element_type=jnp.float32)
                     + b1_ref[...]) * _BN)
    h = jax.nn.relu((jnp.dot(h, w2_ref[...],
                             preferred_element_type=jnp.float32)
                     + b2_ref[...]) * _BN)
    W = jax.nn.relu((jnp.dot(h, w3_ref[...],
                             preferred_element_type=jnp.float32)
                     + b3_ref[...]) * _BN)
    W3 = W.reshape(Sb, K, 16)

    fin = jnp.zeros((Sb, Cout), jnp.float32)
    for w in range(16):
        t_w = jnp.sum(A3 * W3[:, :, w:w + 1], axis=1)
        fin = fin + jnp.dot(t_w, lw_ref[...][w],
                            preferred_element_type=jnp.float32)
    fin = jax.nn.relu((fin + lb_ref[...]) * _BN)
    o_ref[...] = fin[None]


def _sa_layer(xyz_t, pts_t, inv_den, idx, new_xyz, p, Sb):
    B, N, _ = xyz_t.shape
    S, K = idx.shape[1], idx.shape[2]
    Cpts = pts_t.shape[2]
    Cout = p['conv_w'].shape[0]
    Pfull = jnp.concatenate([xyz_t, pts_t, inv_den[:, :, None]], axis=-1)
    Call = Pfull.shape[2]

    cwa = p['conv_w'].T[0:3]
    cwb = p['conv_w'].T[3:]
    cb = p['conv_b'][None, :]
    (w1, b1), (w2, b2), (w3, b3) = p['wn']
    lw = jnp.transpose(p['lin_w'].reshape(Cout, Cout, 16), (2, 1, 0))
    lb = p['lin_b'][None, :]

    grid = (B, S // Sb)
    out = pl.pallas_call(
        functools.partial(_sa_kernel, K, Cpts),
        grid=grid,
        in_specs=[
            pl.BlockSpec((1, N, Call), lambda b, s: (b, 0, 0)),
            pl.BlockSpec((1, Sb, K, 1), lambda b, s: (b, s, 0, 0)),
            pl.BlockSpec((1, Sb, 3), lambda b, s: (b, s, 0)),
            pl.BlockSpec((3, Cout), lambda b, s: (0, 0)),
            pl.BlockSpec((Cpts, Cout), lambda b, s: (0, 0)),
            pl.BlockSpec((1, Cout), lambda b, s: (0, 0)),
            pl.BlockSpec((3, 8), lambda b, s: (0, 0)),
            pl.BlockSpec((1, 8), lambda b, s: (0, 0)),
            pl.BlockSpec((8, 8), lambda b, s: (0, 0)),
            pl.BlockSpec((1, 8), lambda b, s: (0, 0)),
            pl.BlockSpec((8, 16), lambda b, s: (0, 0)),
            pl.BlockSpec((1, 16), lambda b, s: (0, 0)),
            pl.BlockSpec((16, Cout, Cout), lambda b, s: (0, 0, 0)),
            pl.BlockSpec((1, Cout), lambda b, s: (0, 0)),
        ],
        out_specs=pl.BlockSpec((1, Sb, Cout), lambda b, s: (b, s, 0)),
        out_shape=jax.ShapeDtypeStruct((B, S, Cout), jnp.float32),
    )(Pfull, idx[:, :, :, None], new_xyz, cwa, cwb, cb,
      w1.T, b1[None, :], w2.T, b2[None, :], w3.T, b3[None, :], lw, lb)
    return out


def _sa3_kernel(xt_ref, iv_ref, pts_ref,
                cwa_ref, cwb_ref, cb_ref,
                w1_ref, b1_ref, w2_ref, b2_ref, w3_ref, b3_ref,
                lw_ref, lb_ref, o_ref):
    X3 = xt_ref[...][0]
    invden = iv_ref[...][0]
    pts = pts_ref[...][0]
    Cout = cb_ref.shape[1]

    A = jnp.dot(X3, cwa_ref[...], preferred_element_type=jnp.float32)
    A = A + jnp.dot(pts, cwb_ref[...], preferred_element_type=jnp.float32)
    A = jax.nn.relu((A + cb_ref[...]) * _BN)
    A = A * (invden / jnp.max(invden, axis=0, keepdims=True))

    h = jax.nn.relu((jnp.dot(X3, w1_ref[...],
                             preferred_element_type=jnp.float32)
                     + b1_ref[...]) * _BN)
    h = jax.nn.relu((jnp.dot(h, w2_ref[...],
                             preferred_element_type=jnp.float32)
                     + b2_ref[...]) * _BN)
    W = jax.nn.relu((jnp.dot(h, w3_ref[...],
                             preferred_element_type=jnp.float32)
                     + b3_ref[...]) * _BN)

    fin = jnp.zeros((1, Cout), jnp.float32)
    for w in range(16):
        t_w = jnp.sum(A * W[:, w:w + 1], axis=0, keepdims=True)
        fin = fin + jnp.dot(t_w, lw_ref[...][w],
                            preferred_element_type=jnp.float32)
    fin = jax.nn.relu((fin + lb_ref[...]) * _BN)
    o_ref[...] = fin[None]


def _sa3(xyz_t, pts_t, p, bandwidth):
    B, N, _ = xyz_t.shape
    Cpts = pts_t.shape[2]
    Cout = p['conv_w'].shape[0]
    inv_den = _inv_density(xyz_t, bandwidth)[:, :, None]
    cwa = p['conv_w'].T[0:3]
    cwb = p['conv_w'].T[3:]
    (w1, b1), (w2, b2), (w3, b3) = p['wn']
    lw = jnp.transpose(p['lin_w'].reshape(Cout, Cout, 16), (2, 1, 0))
    return pl.pallas_call(
        _sa3_kernel,
        grid=(B,),
        in_specs=[
            pl.BlockSpec((1, N, 3), lambda b: (b, 0, 0)),
            pl.BlockSpec((1, N, 1), lambda b: (b, 0, 0)),
            pl.BlockSpec((1, N, Cpts), lambda b: (b, 0, 0)),
            pl.BlockSpec((3, Cout), lambda b: (0, 0)),
            pl.BlockSpec((Cpts, Cout), lambda b: (0, 0)),
            pl.BlockSpec((1, Cout), lambda b: (0, 0)),
            pl.BlockSpec((3, 8), lambda b: (0, 0)),
            pl.BlockSpec((1, 8), lambda b: (0, 0)),
            pl.BlockSpec((8, 8), lambda b: (0, 0)),
            pl.BlockSpec((1, 8), lambda b: (0, 0)),
            pl.BlockSpec((8, 16), lambda b: (0, 0)),
            pl.BlockSpec((1, 16), lambda b: (0, 0)),
            pl.BlockSpec((16, Cout, Cout), lambda b: (0, 0, 0)),
            pl.BlockSpec((1, Cout), lambda b: (0, 0)),
        ],
        out_specs=pl.BlockSpec((1, 1, Cout), lambda b: (b, 0, 0)),
        out_shape=jax.ShapeDtypeStruct((B, 1, Cout), jnp.float32),
    )(xyz_t, inv_den, pts_t, cwa, cwb, p['conv_b'][None, :],
      w1.T, b1[None, :], w2.T, b2[None, :], w3.T, b3[None, :],
      lw, p['lin_b'][None, :])[:, 0, :]


def _head_kernel(x_ref, w1_ref, b1_ref, g_ref, be_ref, w2_ref, b2_ref, o_ref):
    x = x_ref[...]
    y = jnp.dot(x, w1_ref[...].T, preferred_element_type=jnp.float32)
    y = y + b1_ref[...][None, :]
    mean = jnp.mean(y, axis=1, keepdims=True)
    var = jnp.mean((y - mean) ** 2, axis=1, keepdims=True)
    y = (y - mean) / jnp.sqrt(var + 1e-5)
    y = y * g_ref[...][None, :] + be_ref[...][None, :]
    y = jax.nn.relu(y)
    z = jnp.dot(y, w2_ref[...].T, preferred_element_type=jnp.float32)
    z = z + b2_ref[...][None, :]
    z = z - jax.scipy.special.logsumexp(z, axis=1, keepdims=True)
    o_ref[...] = jnp.broadcast_to(z[:, :, None], o_ref.shape)


def _head(xcat, params):
    B = xcat.shape[0]
    return pl.pallas_call(
        _head_kernel,
        out_shape=jax.ShapeDtypeStruct((B, 2, 1024), jnp.float32),
    )(xcat, params['conv1_w'], params['conv1_b'], params['gn_g'],
      params['gn_b'], params['conv2_w'], params['conv2_b'])


def _sa_stage(xyz_t, pts_t, p, npoint, nsample, bandwidth, Sb):
    fps_idx = _fps(xyz_t, npoint)
    inv_den = _inv_density(xyz_t, bandwidth)
    new_xyz = _index_points(xyz_t, fps_idx)
    dist = _square_distance(new_xyz, xyz_t)
    _, idx = jax.lax.top_k(-dist, nsample)
    out = _sa_layer(xyz_t, pts_t, inv_den, idx, new_xyz, p, Sb)
    return new_xyz, out


def _encode(xyz, params):
    xyz_t = jnp.swapaxes(xyz, 1, 2)
    l1_xyz, l1_pts = _sa_stage(xyz_t, xyz_t, params['sa1'], 512, 32, 0.1, 64)
    l2_xyz, l2_pts = _sa_stage(l1_xyz, l1_pts, params['sa2'], 128, 64, 0.2, 32)
    return _sa3(l2_xyz, l2_pts, params['sa3'], 0.4)


@jax.jit
def _run(xyz, xyz_goal, params):
    feats_x = _encode(xyz, params)
    feats_g = _encode(xyz_goal, params)
    xcat = jnp.concatenate([feats_x, feats_g], axis=-1)
    return _head(xcat, params)


def kernel(xyz, xyz_goal, params):
    return _run(xyz, xyz_goal, params)
